# Initial kernel scaffold; baseline (speedup 1.0000x reference)
import jax
import jax.numpy as jnp
from jax.experimental import pallas as pl


def kernel(x, edge_index, W1, b1, W2, b2, W3, b3, Wf1, bf1, Wf2, bf2, Wf3, bf3):
    raise NotImplementedError("write your pallas kernel here")


# speedup vs baseline: 14.8512x; 14.8512x over previous
import functools

import jax
import jax.numpy as jnp
from jax import lax
from jax.experimental import pallas as pl
from jax.experimental.pallas import tpu as pltpu
from jax.experimental.pallas import tpu_sc as plsc

N_PAD = 10240
NC, NS = 2, 16
NW = NC * NS
SLAB = N_PAD // NS
E_CHUNK = 128
ROW_BLK = 1280
GRID = N_PAD // ROW_BLK


def _make_sc_deg(e_pad):
    e_w = e_pad // NW
    n_chunks = e_w // E_CHUNK
    mesh = plsc.VectorSubcoreMesh(core_axis_name="c", subcore_axis_name="s")

    @functools.partial(
        pl.kernel,
        out_type=jax.ShapeDtypeStruct((NC, N_PAD, 8), jnp.float32),
        mesh=mesh,
        compiler_params=pltpu.CompilerParams(use_tc_tiling_on_sc=False),
        scratch_types=[
            pltpu.VMEM((E_CHUNK,), jnp.int32),
            pltpu.VMEM((E_CHUNK, 8), jnp.float32),
            pltpu.VMEM_SHARED((N_PAD, 8), jnp.float32),
        ],
    )
    def deg_kernel(dst_hbm, ones_hbm, zeros_hbm, out_hbm, dstb, onesb, acc):
        c = lax.axis_index("c")
        s = lax.axis_index("s")
        wid = s * NC + c
        slab = s * SLAB
        pltpu.sync_copy(zeros_hbm.at[pl.ds(slab, SLAB)], acc.at[pl.ds(slab, SLAB)])
        pltpu.sync_copy(ones_hbm, onesb)
        plsc.subcore_barrier()
        ebase = wid * e_w

        def body(i, carry):
            base = ebase + i * E_CHUNK
            pltpu.sync_copy(dst_hbm.at[pl.ds(base, E_CHUNK)], dstb)
            pltpu.sync_copy(onesb, acc.at[dstb], add=True)
            return carry

        lax.fori_loop(0, n_chunks, body, 0)
        plsc.subcore_barrier()
        pltpu.sync_copy(acc.at[pl.ds(slab, SLAB)],
                        out_hbm.at[c, pl.ds(slab, SLAB)])

    return deg_kernel


def _make_sc_agg(feat, e_pad):
    e_w = e_pad // NW
    n_chunks = e_w // E_CHUNK
    mesh = plsc.VectorSubcoreMesh(core_axis_name="c", subcore_axis_name="s")

    @functools.partial(
        pl.kernel,
        out_type=jax.ShapeDtypeStruct((NC, N_PAD, feat), jnp.float32),
        mesh=mesh,
        compiler_params=pltpu.CompilerParams(use_tc_tiling_on_sc=False),
        scratch_types=[
            pltpu.VMEM((E_CHUNK,), jnp.int32),
            pltpu.VMEM((E_CHUNK,), jnp.int32),
            pltpu.VMEM((E_CHUNK, feat), jnp.float32),
            pltpu.VMEM_SHARED((N_PAD, feat), jnp.float32),
            pltpu.SemaphoreType.DMA,
        ],
    )
    def agg_kernel(g_hbm, src_hbm, dst_hbm, zeros_hbm, out_hbm,
                   srcb, dstb, rows, acc, sem):
        c = lax.axis_index("c")
        s = lax.axis_index("s")
        wid = s * NC + c
        slab = s * SLAB
        pltpu.sync_copy(zeros_hbm.at[pl.ds(slab, SLAB)], acc.at[pl.ds(slab, SLAB)])
        plsc.subcore_barrier()
        ebase = wid * e_w

        def body(i, carry):
            base = ebase + i * E_CHUNK
            pltpu.sync_copy(src_hbm.at[pl.ds(base, E_CHUNK)], srcb)
            pltpu.sync_copy(dst_hbm.at[pl.ds(base, E_CHUNK)], dstb)
            pltpu.async_copy(g_hbm.at[srcb], rows, sem).wait()
            pltpu.sync_copy(rows, acc.at[dstb], add=True)
            return carry

        lax.fori_loop(0, n_chunks, body, 0)
        plsc.subcore_barrier()
        pltpu.sync_copy(acc.at[pl.ds(slab, SLAB)],
                        out_hbm.at[c, pl.ds(slab, SLAB)])

    return agg_kernel


def _elu(x):
    return jnp.where(x > 0, x, jnp.exp(jnp.minimum(x, 0.0)) - 1.0)


def _tc_layer0(dacc, x, w1):
    f = w1.shape[1]

    def body(dref, xref, wref, gout, dout):
        d = dref[...]
        deg = d[0, :, 0:1] + d[1, :, 0:1]
        di = lax.rsqrt(jnp.maximum(deg, 1.0))
        t = jnp.dot(xref[...], wref[...], preferred_element_type=jnp.float32)
        gout[...] = di * t
        dout[...] = jnp.broadcast_to(di, (ROW_BLK, 8))

    return pl.pallas_call(
        body,
        grid=(GRID,),
        in_specs=[
            pl.BlockSpec((2, ROW_BLK, 8), lambda i: (0, i, 0)),
            pl.BlockSpec((ROW_BLK, 128), lambda i: (i, 0)),
            pl.BlockSpec((128, f), lambda i: (0, 0)),
        ],
        out_specs=[
            pl.BlockSpec((ROW_BLK, f), lambda i: (i, 0)),
            pl.BlockSpec((ROW_BLK, 8), lambda i: (i, 0)),
        ],
        out_shape=[
            jax.ShapeDtypeStruct((N_PAD, f), jnp.float32),
            jax.ShapeDtypeStruct((N_PAD, 8), jnp.float32),
        ],
    )(dacc, x, w1)


def _tc_mid(sacc, g_prev, dinv, b, w_next):
    f = g_prev.shape[1]
    fn = w_next.shape[1]

    def body(aref, gref, dref, bref, wref, out):
        a = aref[...]
        di = dref[...][:, 0:1]
        h = _elu(di * (a[0] + a[1] + gref[...]) + bref[...])
        out[...] = di * jnp.dot(h, wref[...], preferred_element_type=jnp.float32)

    return pl.pallas_call(
        body,
        grid=(GRID,),
        in_specs=[
            pl.BlockSpec((2, ROW_BLK, f), lambda i: (0, i, 0)),
            pl.BlockSpec((ROW_BLK, f), lambda i: (i, 0)),
            pl.BlockSpec((ROW_BLK, 8), lambda i: (i, 0)),
            pl.BlockSpec((1, f), lambda i: (0, 0)),
            pl.BlockSpec((f, fn), lambda i: (0, 0)),
        ],
        out_specs=pl.BlockSpec((ROW_BLK, fn), lambda i: (i, 0)),
        out_shape=jax.ShapeDtypeStruct((N_PAD, fn), jnp.float32),
    )(sacc, g_prev, dinv, b, w_next)


def _tc_head(sacc, g3, dinv, b3, wf1, bf1, wf2, bf2, wf3, bf3):
    f = g3.shape[1]

    def body(aref, gref, dref, b3ref, w1ref, b1ref, w2ref, b2ref, w3ref,
             b3fref, out):
        a = aref[...]
        di = dref[...][:, 0:1]
        h = _elu(di * (a[0] + a[1] + gref[...]) + b3ref[...])
        z = _elu(jnp.dot(h, w1ref[...], preferred_element_type=jnp.float32)
                 + b1ref[...])
        z = _elu(jnp.dot(z, w2ref[...], preferred_element_type=jnp.float32)
                 + b2ref[...])
        z = (jnp.dot(z, w3ref[...], preferred_element_type=jnp.float32)
             + b3fref[...])
        m = jnp.max(z, axis=1, keepdims=True)
        lse = m + jnp.log(jnp.sum(jnp.exp(z - m), axis=1, keepdims=True))
        out[...] = z - lse

    return pl.pallas_call(
        body,
        grid=(GRID,),
        in_specs=[
            pl.BlockSpec((2, ROW_BLK, f), lambda i: (0, i, 0)),
            pl.BlockSpec((ROW_BLK, f), lambda i: (i, 0)),
            pl.BlockSpec((ROW_BLK, 8), lambda i: (i, 0)),
            pl.BlockSpec((1, f), lambda i: (0, 0)),
            pl.BlockSpec(wf1.shape, lambda i: (0, 0)),
            pl.BlockSpec((1, wf1.shape[1]), lambda i: (0, 0)),
            pl.BlockSpec(wf2.shape, lambda i: (0, 0)),
            pl.BlockSpec((1, wf2.shape[1]), lambda i: (0, 0)),
            pl.BlockSpec(wf3.shape, lambda i: (0, 0)),
            pl.BlockSpec((1, wf3.shape[1]), lambda i: (0, 0)),
        ],
        out_specs=pl.BlockSpec((ROW_BLK, wf3.shape[1]), lambda i: (i, 0)),
        out_shape=jax.ShapeDtypeStruct((N_PAD, wf3.shape[1]), jnp.float32),
    )(sacc, g3, dinv, b3, wf1, bf1, wf2, bf2, wf3, bf3)


def kernel(x, edge_index, W1, b1, W2, b2, W3, b3,
           Wf1, bf1, Wf2, bf2, Wf3, bf3):
    n = x.shape[0]
    e = edge_index.shape[1]
    e_pad = ((e + NW * E_CHUNK - 1) // (NW * E_CHUNK)) * (NW * E_CHUNK)

    x_pad = jnp.zeros((N_PAD, x.shape[1]), x.dtype).at[:n].set(x)
    src = jnp.concatenate(
        [edge_index[0].astype(jnp.int32),
         jnp.zeros((e_pad - e,), jnp.int32)])
    dst = jnp.concatenate(
        [edge_index[1].astype(jnp.int32),
         jnp.full((e_pad - e,), n, jnp.int32)])

    ones8 = jnp.ones((E_CHUNK, 8), jnp.float32)
    zeros8 = jnp.zeros((N_PAD, 8), jnp.float32)

    dacc = _make_sc_deg(e_pad)(dst, ones8, zeros8)
    g1, dinv = _tc_layer0(dacc, x_pad, W1)

    s1 = _make_sc_agg(16, e_pad)(g1, src, dst, jnp.zeros((N_PAD, 16), jnp.float32))
    g2 = _tc_mid(s1, g1, dinv, b1.reshape(1, -1), W2)

    s2 = _make_sc_agg(32, e_pad)(g2, src, dst, jnp.zeros((N_PAD, 32), jnp.float32))
    g3 = _tc_mid(s2, g2, dinv, b2.reshape(1, -1), W3)

    s3 = _make_sc_agg(64, e_pad)(g3, src, dst, jnp.zeros((N_PAD, 64), jnp.float32))
    out = _tc_head(s3, g3, dinv, b3.reshape(1, -1),
                   Wf1, bf1.reshape(1, -1), Wf2, bf2.reshape(1, -1),
                   Wf3, bf3.reshape(1, -1))
    return out[:n]


# speedup vs baseline: 20.2492x; 1.3635x over previous
import functools

import jax
import jax.numpy as jnp
from jax import lax
from jax.experimental import pallas as pl
from jax.experimental.pallas import tpu as pltpu
from jax.experimental.pallas import tpu_sc as plsc

N_PAD = 10240
NC, NS = 2, 16
NW = NC * NS
SLAB = N_PAD // NS
E_CHUNK = 128
ROW_BLK = 1280
GRID = N_PAD // ROW_BLK


NBUF = 4


def _make_sc_deg(e_pad):
    e_w = e_pad // NW
    n_chunks = e_w // E_CHUNK
    mesh = plsc.VectorSubcoreMesh(core_axis_name="c", subcore_axis_name="s")

    @functools.partial(
        pl.kernel,
        out_type=jax.ShapeDtypeStruct((NC, N_PAD, 8), jnp.float32),
        mesh=mesh,
        compiler_params=pltpu.CompilerParams(use_tc_tiling_on_sc=False),
        scratch_types=[
            pltpu.VMEM((e_pad // NW // E_CHUNK, E_CHUNK), jnp.int32),
            pltpu.VMEM((E_CHUNK, 8), jnp.float32),
            pltpu.VMEM_SHARED((N_PAD, 8), jnp.float32),
            pltpu.SemaphoreType.DMA((NBUF,)),
        ],
    )
    def deg_kernel(dst_hbm, ones_hbm, zeros_hbm, out_hbm, dstv, onesb, acc, sems):
        c = lax.axis_index("c")
        s = lax.axis_index("s")
        wid = s * NC + c
        slab = s * SLAB
        pltpu.sync_copy(zeros_hbm.at[pl.ds(slab, SLAB)], acc.at[pl.ds(slab, SLAB)])
        pltpu.sync_copy(ones_hbm, onesb)
        pltpu.sync_copy(dst_hbm.at[pl.ds(wid * n_chunks, n_chunks), :], dstv)
        plsc.subcore_barrier()

        def body(i, carry):
            descs = []
            for b in range(NBUF):
                j = i * NBUF + b
                descs.append(
                    pltpu.async_copy(onesb, acc.at[dstv.at[j]], sems.at[b],
                                     add=True))
            for d in descs:
                d.wait()
            return carry

        lax.fori_loop(0, n_chunks // NBUF, body, 0)
        plsc.subcore_barrier()
        pltpu.sync_copy(acc.at[pl.ds(slab, SLAB)],
                        out_hbm.at[c, pl.ds(slab, SLAB)])

    return deg_kernel


def _make_sc_agg(feat, e_pad):
    e_w = e_pad // NW
    n_chunks = e_w // E_CHUNK
    mesh = plsc.VectorSubcoreMesh(core_axis_name="c", subcore_axis_name="s")

    @functools.partial(
        pl.kernel,
        out_type=jax.ShapeDtypeStruct((NC, N_PAD, feat), jnp.float32),
        mesh=mesh,
        compiler_params=pltpu.CompilerParams(use_tc_tiling_on_sc=False),
        scratch_types=[
            pltpu.VMEM((e_pad // NW // E_CHUNK, E_CHUNK), jnp.int32),
            pltpu.VMEM((e_pad // NW // E_CHUNK, E_CHUNK), jnp.int32),
            pltpu.VMEM((NBUF, E_CHUNK, feat), jnp.float32),
            pltpu.VMEM_SHARED((N_PAD, feat), jnp.float32),
            pltpu.SemaphoreType.DMA((NBUF,)),
            pltpu.SemaphoreType.DMA((NBUF,)),
        ],
    )
    def agg_kernel(g_hbm, src_hbm, dst_hbm, zeros_hbm, out_hbm,
                   srcv, dstv, rows, acc, semg, sems):
        c = lax.axis_index("c")
        s = lax.axis_index("s")
        wid = s * NC + c
        slab = s * SLAB
        pltpu.sync_copy(zeros_hbm.at[pl.ds(slab, SLAB)], acc.at[pl.ds(slab, SLAB)])
        pltpu.sync_copy(src_hbm.at[pl.ds(wid * n_chunks, n_chunks), :], srcv)
        pltpu.sync_copy(dst_hbm.at[pl.ds(wid * n_chunks, n_chunks), :], dstv)
        plsc.subcore_barrier()

        def gather(j, b):
            return pltpu.async_copy(g_hbm.at[srcv.at[j]], rows.at[b],
                                    semg.at[b])

        def scatter(j, b):
            return pltpu.async_copy(rows.at[b], acc.at[dstv.at[j]],
                                    sems.at[b], add=True)

        for b in range(NBUF):
            gather(b, b)

        def body(i, carry):
            base = i * NBUF
            for b in range(NBUF):
                pltpu.make_async_copy(g_hbm.at[srcv.at[0]], rows.at[b],
                                      semg.at[b]).wait()
                scatter(base + b, b)
            for b in range(NBUF):
                pltpu.make_async_copy(rows.at[b], acc.at[dstv.at[0]],
                                      sems.at[b]).wait()
                nxt = jnp.minimum(base + NBUF + b, n_chunks - 1)
                gather(nxt, b)
            return carry

        lax.fori_loop(0, n_chunks // NBUF, body, 0)
        for b in range(NBUF):
            pltpu.make_async_copy(g_hbm.at[srcv.at[0]], rows.at[b],
                                  semg.at[b]).wait()
        plsc.subcore_barrier()
        pltpu.sync_copy(acc.at[pl.ds(slab, SLAB)],
                        out_hbm.at[c, pl.ds(slab, SLAB)])

    return agg_kernel


def _elu(x):
    return jnp.where(x > 0, x, jnp.exp(jnp.minimum(x, 0.0)) - 1.0)


def _tc_layer0(dacc, x, w1):
    f = w1.shape[1]

    def body(dref, xref, wref, gout, dout):
        d = dref[...]
        deg = d[0, :, 0:1] + d[1, :, 0:1]
        di = lax.rsqrt(jnp.maximum(deg, 1.0))
        t = jnp.dot(xref[...], wref[...], preferred_element_type=jnp.float32)
        gout[...] = di * t
        dout[...] = jnp.broadcast_to(di, (ROW_BLK, 8))

    return pl.pallas_call(
        body,
        grid=(GRID,),
        in_specs=[
            pl.BlockSpec((2, ROW_BLK, 8), lambda i: (0, i, 0)),
            pl.BlockSpec((ROW_BLK, 128), lambda i: (i, 0)),
            pl.BlockSpec((128, f), lambda i: (0, 0)),
        ],
        out_specs=[
            pl.BlockSpec((ROW_BLK, f), lambda i: (i, 0)),
            pl.BlockSpec((ROW_BLK, 8), lambda i: (i, 0)),
        ],
        out_shape=[
            jax.ShapeDtypeStruct((N_PAD, f), jnp.float32),
            jax.ShapeDtypeStruct((N_PAD, 8), jnp.float32),
        ],
    )(dacc, x, w1)


def _tc_mid(sacc, g_prev, dinv, b, w_next):
    f = g_prev.shape[1]
    fn = w_next.shape[1]

    def body(aref, gref, dref, bref, wref, out):
        a = aref[...]
        di = dref[...][:, 0:1]
        h = _elu(di * (a[0] + a[1] + gref[...]) + bref[...])
        out[...] = di * jnp.dot(h, wref[...], preferred_element_type=jnp.float32)

    return pl.pallas_call(
        body,
        grid=(GRID,),
        in_specs=[
            pl.BlockSpec((2, ROW_BLK, f), lambda i: (0, i, 0)),
            pl.BlockSpec((ROW_BLK, f), lambda i: (i, 0)),
            pl.BlockSpec((ROW_BLK, 8), lambda i: (i, 0)),
            pl.BlockSpec((1, f), lambda i: (0, 0)),
            pl.BlockSpec((f, fn), lambda i: (0, 0)),
        ],
        out_specs=pl.BlockSpec((ROW_BLK, fn), lambda i: (i, 0)),
        out_shape=jax.ShapeDtypeStruct((N_PAD, fn), jnp.float32),
    )(sacc, g_prev, dinv, b, w_next)


def _tc_head(sacc, g3, dinv, b3, wf1, bf1, wf2, bf2, wf3, bf3):
    f = g3.shape[1]

    def body(aref, gref, dref, b3ref, w1ref, b1ref, w2ref, b2ref, w3ref,
             b3fref, out):
        a = aref[...]
        di = dref[...][:, 0:1]
        h = _elu(di * (a[0] + a[1] + gref[...]) + b3ref[...])
        z = _elu(jnp.dot(h, w1ref[...], preferred_element_type=jnp.float32)
                 + b1ref[...])
        z = _elu(jnp.dot(z, w2ref[...], preferred_element_type=jnp.float32)
                 + b2ref[...])
        z = (jnp.dot(z, w3ref[...], preferred_element_type=jnp.float32)
             + b3fref[...])
        m = jnp.max(z, axis=1, keepdims=True)
        lse = m + jnp.log(jnp.sum(jnp.exp(z - m), axis=1, keepdims=True))
        out[...] = z - lse

    return pl.pallas_call(
        body,
        grid=(GRID,),
        in_specs=[
            pl.BlockSpec((2, ROW_BLK, f), lambda i: (0, i, 0)),
            pl.BlockSpec((ROW_BLK, f), lambda i: (i, 0)),
            pl.BlockSpec((ROW_BLK, 8), lambda i: (i, 0)),
            pl.BlockSpec((1, f), lambda i: (0, 0)),
            pl.BlockSpec(wf1.shape, lambda i: (0, 0)),
            pl.BlockSpec((1, wf1.shape[1]), lambda i: (0, 0)),
            pl.BlockSpec(wf2.shape, lambda i: (0, 0)),
            pl.BlockSpec((1, wf2.shape[1]), lambda i: (0, 0)),
            pl.BlockSpec(wf3.shape, lambda i: (0, 0)),
            pl.BlockSpec((1, wf3.shape[1]), lambda i: (0, 0)),
        ],
        out_specs=pl.BlockSpec((ROW_BLK, wf3.shape[1]), lambda i: (i, 0)),
        out_shape=jax.ShapeDtypeStruct((N_PAD, wf3.shape[1]), jnp.float32),
    )(sacc, g3, dinv, b3, wf1, bf1, wf2, bf2, wf3, bf3)


def kernel(x, edge_index, W1, b1, W2, b2, W3, b3,
           Wf1, bf1, Wf2, bf2, Wf3, bf3):
    n = x.shape[0]
    e = edge_index.shape[1]
    quantum = NW * E_CHUNK * NBUF
    e_pad = ((e + quantum - 1) // quantum) * quantum

    x_pad = jnp.zeros((N_PAD, x.shape[1]), x.dtype).at[:n].set(x)
    src = jnp.concatenate(
        [edge_index[0].astype(jnp.int32),
         jnp.zeros((e_pad - e,), jnp.int32)]).reshape(-1, E_CHUNK)
    dst = jnp.concatenate(
        [edge_index[1].astype(jnp.int32),
         jnp.full((e_pad - e,), n, jnp.int32)]
        ).reshape(-1, E_CHUNK)

    ones8 = jnp.ones((E_CHUNK, 8), jnp.float32)
    zeros8 = jnp.zeros((N_PAD, 8), jnp.float32)

    dacc = _make_sc_deg(e_pad)(dst, ones8, zeros8)
    g1, dinv = _tc_layer0(dacc, x_pad, W1)

    s1 = _make_sc_agg(16, e_pad)(g1, src, dst, jnp.zeros((N_PAD, 16), jnp.float32))
    g2 = _tc_mid(s1, g1, dinv, b1.reshape(1, -1), W2)

    s2 = _make_sc_agg(32, e_pad)(g2, src, dst, jnp.zeros((N_PAD, 32), jnp.float32))
    g3 = _tc_mid(s2, g2, dinv, b2.reshape(1, -1), W3)

    s3 = _make_sc_agg(64, e_pad)(g3, src, dst, jnp.zeros((N_PAD, 64), jnp.float32))
    out = _tc_head(s3, g3, dinv, b3.reshape(1, -1),
                   Wf1, bf1.reshape(1, -1), Wf2, bf2.reshape(1, -1),
                   Wf3, bf3.reshape(1, -1))
    return out[:n]
